# initial kernel scaffold (unmeasured)
import jax
import jax.numpy as jnp
from jax import lax
from jax.experimental import pallas as pl
from jax.experimental.pallas import tpu as pltpu

N_DEV = 8
CH = 512


def kernel(x, w_mat):
    m, _ = x.shape
    _, n = w_mat.shape

    x = x.astype(jnp.bfloat16)
    w_mat = w_mat.astype(jnp.bfloat16)

    def body(x_ref, w_ref, out_ref, rbuf, sbuf, send_sems, recv_sems,
             credit, out_sems):
        i = lax.axis_index("i")
        left = lax.rem(i - 1 + N_DEV, N_DEV)
        right = lax.rem(i + 1, N_DEV)

        barrier = pltpu.get_barrier_semaphore()
        for nbr in (left, right):
            pl.semaphore_signal(barrier, 1, device_id=(nbr,),
                                device_id_type=pl.DeviceIdType.MESH)
        pl.semaphore_wait(barrier, 2)

        pl.semaphore_signal(credit, 2, device_id=(left,),
                            device_id_type=pl.DeviceIdType.MESH)

        def partial_chunk(c):
            rows = pl.ds(c * CH, CH)
            return jnp.dot(x_ref[rows, :], w_ref[:, :],
                           preferred_element_type=jnp.float32)

        gc = 0.7978845608028654

        def gelu(y):
            return 0.5 * y * (1.0 + jnp.tanh(gc * (y + 0.044715 * y * y * y)))

        pending = [None, None]

        def fill_sbuf(slot, value):
            if pending[slot] is not None:
                pending[slot].wait()
                pending[slot] = None
            sbuf[slot, :, :] = value

        def store_out(slot, c):
            cp = pltpu.make_async_copy(
                sbuf.at[slot], out_ref.at[pl.ds(c * CH, CH), :],
                out_sems.at[slot])
            cp.start()
            pending[slot] = cp

        def ring_send(step):
            slot = step % 2
            pl.semaphore_wait(credit, 1)
            rdma = pltpu.make_async_remote_copy(
                src_ref=sbuf.at[slot], dst_ref=rbuf.at[slot],
                send_sem=send_sems.at[slot], recv_sem=recv_sems.at[slot],
                device_id=(right,), device_id_type=pl.DeviceIdType.MESH)
            rdma.start()
            return rdma

        fill_sbuf(0, partial_chunk(i).astype(jnp.bfloat16))
        total = None
        for s in range(N_DEV - 1):
            slot = s % 2
            rdma = ring_send(s)
            c_recv = lax.rem(i - 1 - s + 2 * N_DEV, N_DEV)
            part = partial_chunk(c_recv)
            rdma.wait()
            total = part + rbuf[slot, :, :].astype(jnp.float32)
            pl.semaphore_signal(credit, 1, device_id=(left,),
                                device_id_type=pl.DeviceIdType.MESH)
            if s < N_DEV - 2:
                fill_sbuf((s + 1) % 2, total.astype(jnp.bfloat16))

        c_mine = lax.rem(i + 1, N_DEV)
        fill_sbuf(1, gelu(total).astype(jnp.bfloat16))
        store_out(1, c_mine)

        for h in range(N_DEV - 1):
            t = N_DEV - 1 + h
            slot = t % 2
            rdma = ring_send(t)
            rdma.wait()
            c_recv = lax.rem(i - h + 2 * N_DEV, N_DEV)
            nslot = (t + 1) % 2
            fill_sbuf(nslot, rbuf[slot, :, :])
            pl.semaphore_signal(credit, 1, device_id=(left,),
                                device_id_type=pl.DeviceIdType.MESH)
            store_out(nslot, c_recv)

        pl.semaphore_wait(credit, 2)
        for slot in (0, 1):
            if pending[slot] is not None:
                pending[slot].wait()
                pending[slot] = None

    return pl.pallas_call(
        body,
        out_shape=jax.ShapeDtypeStruct((m, n), jnp.bfloat16),
        in_specs=[pl.BlockSpec(memory_space=pltpu.VMEM),
                  pl.BlockSpec(memory_space=pltpu.VMEM)],
        out_specs=pl.BlockSpec(memory_space=pltpu.ANY),
        scratch_shapes=[
            pltpu.VMEM((2, CH, n), jnp.bfloat16),
            pltpu.VMEM((2, CH, n), jnp.bfloat16),
            pltpu.SemaphoreType.DMA((2,)),
            pltpu.SemaphoreType.DMA((2,)),
            pltpu.SemaphoreType.REGULAR,
            pltpu.SemaphoreType.DMA((2,)),
        ],
        compiler_params=pltpu.CompilerParams(collective_id=0),
    )(x, w_mat)


# baseline (device time: 1411230 ns/iter reference)
import jax
import jax.numpy as jnp
from jax import lax
from jax.experimental import pallas as pl
from jax.experimental.pallas import tpu as pltpu

N_DEV = 8
CH = 512


def kernel(x, w_mat):
    m, _ = x.shape
    _, n = w_mat.shape

    x = x.astype(jnp.bfloat16)
    w_mat = w_mat.astype(jnp.bfloat16)

    def body(x_ref, w_ref, out_ref, rbuf, sbuf, send_sems, recv_sems,
             credit, out_sems):
        i = lax.axis_index("i")
        left = lax.rem(i - 1 + N_DEV, N_DEV)
        right = lax.rem(i + 1, N_DEV)

        barrier = pltpu.get_barrier_semaphore()
        for nbr in (left, right):
            pl.semaphore_signal(barrier, 1, device_id=(nbr,),
                                device_id_type=pl.DeviceIdType.MESH)
        pl.semaphore_wait(barrier, 2)

        pl.semaphore_signal(credit, 2, device_id=(left,),
                            device_id_type=pl.DeviceIdType.MESH)

        def partial_chunk(c):
            rows = pl.ds(c * CH, CH)
            return jnp.dot(x_ref[rows, :], w_ref[:, :],
                           preferred_element_type=jnp.float32)

        gc = 0.7978845608028654

        def gelu(y):
            return 0.5 * y * (1.0 + jnp.tanh(gc * (y + 0.044715 * y * y * y)))

        pending = [None, None]

        def fill_sbuf(slot, value):
            if pending[slot] is not None:
                pending[slot].wait()
                pending[slot] = None
            sbuf[slot, :, :] = value

        def store_out(slot, c):
            cp = pltpu.make_async_copy(
                sbuf.at[slot], out_ref.at[pl.ds(c * CH, CH), :],
                out_sems.at[slot])
            cp.start()
            pending[slot] = cp

        def ring_send(step):
            slot = step % 2
            pl.semaphore_wait(credit, 1)
            rdma = pltpu.make_async_remote_copy(
                src_ref=sbuf.at[slot], dst_ref=rbuf.at[slot],
                send_sem=send_sems.at[slot], recv_sem=recv_sems.at[slot],
                device_id=(right,), device_id_type=pl.DeviceIdType.MESH)
            rdma.start()
            return rdma

        fill_sbuf(0, partial_chunk(i).astype(jnp.bfloat16))
        total = None
        for s in range(N_DEV - 1):
            slot = s % 2
            rdma = ring_send(s)
            c_recv = lax.rem(i - 1 - s + 2 * N_DEV, N_DEV)
            part = partial_chunk(c_recv)
            rdma.wait()
            total = part + rbuf[slot, :, :].astype(jnp.float32)
            pl.semaphore_signal(credit, 1, device_id=(left,),
                                device_id_type=pl.DeviceIdType.MESH)
            if s < N_DEV - 2:
                fill_sbuf((s + 1) % 2, total.astype(jnp.bfloat16))

        c_mine = lax.rem(i + 1, N_DEV)
        fill_sbuf(1, gelu(total).astype(jnp.bfloat16))
        store_out(1, c_mine)

        for h in range(N_DEV - 1):
            t = N_DEV - 1 + h
            slot = t % 2
            rdma = ring_send(t)
            rdma.wait()
            c_recv = lax.rem(i - h + 2 * N_DEV, N_DEV)
            nslot = (t + 1) % 2
            fill_sbuf(nslot, rbuf[slot, :, :])
            pl.semaphore_signal(credit, 1, device_id=(left,),
                                device_id_type=pl.DeviceIdType.MESH)
            store_out(nslot, c_recv)

        pl.semaphore_wait(credit, 2)
        for slot in (0, 1):
            if pending[slot] is not None:
                pending[slot].wait()
                pending[slot] = None

    return pl.pallas_call(
        body,
        out_shape=jax.ShapeDtypeStruct((m, n), jnp.bfloat16),
        in_specs=[pl.BlockSpec(memory_space=pltpu.VMEM),
                  pl.BlockSpec(memory_space=pltpu.VMEM)],
        out_specs=pl.BlockSpec(memory_space=pl.ANY),
        scratch_shapes=[
            pltpu.VMEM((2, CH, n), jnp.bfloat16),
            pltpu.VMEM((2, CH, n), jnp.bfloat16),
            pltpu.SemaphoreType.DMA((2,)),
            pltpu.SemaphoreType.DMA((2,)),
            pltpu.SemaphoreType.REGULAR,
            pltpu.SemaphoreType.DMA((2,)),
        ],
        compiler_params=pltpu.CompilerParams(
            collective_id=0, vmem_limit_bytes=100 * 1024 * 1024),
    )(x, w_mat)


# device time: 783759 ns/iter; 1.8006x vs baseline; 1.8006x over previous
import jax
import jax.numpy as jnp
from jax import lax
from jax.experimental import pallas as pl
from jax.experimental.pallas import tpu as pltpu

N_DEV = 8
CH = 512
HH = CH // 2


def kernel(x, w_mat):
    m, _ = x.shape
    _, n = w_mat.shape

    x = x.astype(jnp.bfloat16)
    w_mat = w_mat.astype(jnp.bfloat16)

    def body(x_ref, w_ref, out_ref,
             rbuf_r, sbuf_r, ssem_r, rsem_r, credit_r, osem_r,
             rbuf_l, sbuf_l, ssem_l, rsem_l, credit_l, osem_l):
        i = lax.axis_index("i")
        left = lax.rem(i - 1 + N_DEV, N_DEV)
        right = lax.rem(i + 1, N_DEV)

        barrier = pltpu.get_barrier_semaphore()
        for nbr in (left, right):
            pl.semaphore_signal(barrier, 1, device_id=(nbr,),
                                device_id_type=pl.DeviceIdType.MESH)
        pl.semaphore_wait(barrier, 2)

        class Ring:

            def __init__(self, d, rbuf, sbuf, ssem, rsem, credit, osem,
                         dst, sender, row_off):
                self.d = d
                self.rbuf, self.sbuf = rbuf, sbuf
                self.ssem, self.rsem = ssem, rsem
                self.credit, self.osem = credit, osem
                self.dst, self.sender = dst, sender
                self.row_off = row_off
                self.pending = [None, None]

            def seed(self):
                pl.semaphore_signal(self.credit, 2, device_id=(self.sender,),
                                    device_id_type=pl.DeviceIdType.MESH)

            def give_credit(self):
                pl.semaphore_signal(self.credit, 1, device_id=(self.sender,),
                                    device_id_type=pl.DeviceIdType.MESH)

            def partial(self, c):
                rows = pl.ds(c * CH + self.row_off, HH)
                return jnp.dot(x_ref[rows, :], w_ref[:, :],
                               preferred_element_type=jnp.float32)

            def fill_sbuf(self, slot, value):
                if self.pending[slot] is not None:
                    self.pending[slot].wait()
                    self.pending[slot] = None
                self.sbuf[slot, :, :] = value

            def store_out(self, slot, c):
                cp = pltpu.make_async_copy(
                    self.sbuf.at[slot],
                    out_ref.at[pl.ds(c * CH + self.row_off, HH), :],
                    self.osem.at[slot])
                cp.start()
                self.pending[slot] = cp

            def send(self, step):
                slot = step % 2
                pl.semaphore_wait(self.credit, 1)
                rdma = pltpu.make_async_remote_copy(
                    src_ref=self.sbuf.at[slot], dst_ref=self.rbuf.at[slot],
                    send_sem=self.ssem.at[slot], recv_sem=self.rsem.at[slot],
                    device_id=(self.dst,),
                    device_id_type=pl.DeviceIdType.MESH)
                rdma.start()
                return rdma

            def rs_recv_chunk(self, s):
                return lax.rem(i - self.d * (1 + s) + 2 * N_DEV, N_DEV)

            def own_chunk(self):
                return lax.rem(i + self.d + N_DEV, N_DEV)

            def ag_recv_chunk(self, h):
                return lax.rem(i - self.d * h + 2 * N_DEV, N_DEV)

        rings = [
            Ring(+1, rbuf_r, sbuf_r, ssem_r, rsem_r, credit_r, osem_r,
                 dst=right, sender=left, row_off=0),
            Ring(-1, rbuf_l, sbuf_l, ssem_l, rsem_l, credit_l, osem_l,
                 dst=left, sender=right, row_off=HH),
        ]

        for r in rings:
            r.seed()

        gc = 0.7978845608028654

        def gelu(y):
            return 0.5 * y * (1.0 + jnp.tanh(gc * (y + 0.044715 * y * y * y)))

        for r in rings:
            r.fill_sbuf(0, r.partial(i).astype(jnp.bfloat16))
        totals = [None, None]
        for s in range(N_DEV - 1):
            slot = s % 2
            rdmas = [r.send(s) for r in rings]
            parts = [r.partial(r.rs_recv_chunk(s)) for r in rings]
            for k, r in enumerate(rings):
                rdmas[k].wait()
                totals[k] = parts[k] + r.rbuf[slot, :, :].astype(jnp.float32)
                r.give_credit()
                if s < N_DEV - 2:
                    r.fill_sbuf((s + 1) % 2, totals[k].astype(jnp.bfloat16))

        for k, r in enumerate(rings):
            r.fill_sbuf(1, gelu(totals[k]).astype(jnp.bfloat16))
            r.store_out(1, r.own_chunk())

        for h in range(N_DEV - 1):
            t = N_DEV - 1 + h
            slot = t % 2
            nslot = (t + 1) % 2
            rdmas = [r.send(t) for r in rings]
            for k, r in enumerate(rings):
                rdmas[k].wait()
                r.fill_sbuf(nslot, r.rbuf[slot, :, :])
                r.give_credit()
                r.store_out(nslot, r.ag_recv_chunk(h))

        for r in rings:
            pl.semaphore_wait(r.credit, 2)
        for r in rings:
            for slot in (0, 1):
                if r.pending[slot] is not None:
                    r.pending[slot].wait()
                    r.pending[slot] = None

    ring_scratch = [
        pltpu.VMEM((2, HH, n), jnp.bfloat16),
        pltpu.VMEM((2, HH, n), jnp.bfloat16),
        pltpu.SemaphoreType.DMA((2,)),
        pltpu.SemaphoreType.DMA((2,)),
        pltpu.SemaphoreType.REGULAR,
        pltpu.SemaphoreType.DMA((2,)),
    ]
    return pl.pallas_call(
        body,
        out_shape=jax.ShapeDtypeStruct((m, n), jnp.bfloat16),
        in_specs=[pl.BlockSpec(memory_space=pltpu.VMEM),
                  pl.BlockSpec(memory_space=pltpu.VMEM)],
        out_specs=pl.BlockSpec(memory_space=pl.ANY),
        scratch_shapes=ring_scratch + ring_scratch,
        compiler_params=pltpu.CompilerParams(
            collective_id=0, vmem_limit_bytes=100 * 1024 * 1024),
    )(x, w_mat)


# device time: 753453 ns/iter; 1.8730x vs baseline; 1.0402x over previous
import jax
import jax.numpy as jnp
from jax import lax
from jax.experimental import pallas as pl
from jax.experimental.pallas import tpu as pltpu

N_DEV = 8
CH = 512
HH = CH // 2


def kernel(x, w_mat):
    m, _ = x.shape
    _, n = w_mat.shape

    x = x.astype(jnp.bfloat16)
    w_mat = w_mat.astype(jnp.bfloat16)

    def body(x_ref, w_ref, out_ref,
             rbuf_r, sbuf_r, ssem_r, rsem_r, credit_r, osem_r,
             rbuf_l, sbuf_l, ssem_l, rsem_l, credit_l, osem_l):
        i = lax.axis_index("i")
        left = lax.rem(i - 1 + N_DEV, N_DEV)
        right = lax.rem(i + 1, N_DEV)

        barrier = pltpu.get_barrier_semaphore()
        for nbr in (left, right):
            pl.semaphore_signal(barrier, 1, device_id=(nbr,),
                                device_id_type=pl.DeviceIdType.MESH)
        pl.semaphore_wait(barrier, 2)

        class Ring:

            def __init__(self, d, rbuf, sbuf, ssem, rsem, credit, osem,
                         dst, sender, row_off):
                self.d = d
                self.rbuf, self.sbuf = rbuf, sbuf
                self.ssem, self.rsem = ssem, rsem
                self.credit, self.osem = credit, osem
                self.dst, self.sender = dst, sender
                self.row_off = row_off
                self.pending = [None, None]

            def seed(self):
                pl.semaphore_signal(self.credit, 2, device_id=(self.sender,),
                                    device_id_type=pl.DeviceIdType.MESH)

            def give_credit(self):
                pl.semaphore_signal(self.credit, 1, device_id=(self.sender,),
                                    device_id_type=pl.DeviceIdType.MESH)

            def partial(self, c):
                rows = pl.ds(c * CH + self.row_off, HH)
                return jnp.dot(x_ref[rows, :], w_ref[:, :],
                               preferred_element_type=jnp.float32
                               ).astype(jnp.bfloat16)

            def fill_sbuf(self, slot, value):
                if self.pending[slot] is not None:
                    self.pending[slot].wait()
                    self.pending[slot] = None
                self.sbuf[slot, :, :] = value

            def store_out(self, slot, c):
                cp = pltpu.make_async_copy(
                    self.sbuf.at[slot],
                    out_ref.at[pl.ds(c * CH + self.row_off, HH), :],
                    self.osem.at[slot])
                cp.start()
                self.pending[slot] = cp

            def send(self, step):
                slot = step % 2
                pl.semaphore_wait(self.credit, 1)
                rdma = pltpu.make_async_remote_copy(
                    src_ref=self.sbuf.at[slot], dst_ref=self.rbuf.at[slot],
                    send_sem=self.ssem.at[slot], recv_sem=self.rsem.at[slot],
                    device_id=(self.dst,),
                    device_id_type=pl.DeviceIdType.MESH)
                rdma.start()
                return rdma

            def rs_recv_chunk(self, s):
                return lax.rem(i - self.d * (1 + s) + 2 * N_DEV, N_DEV)

            def own_chunk(self):
                return lax.rem(i + self.d + N_DEV, N_DEV)

            def ag_recv_chunk(self, h):
                return lax.rem(i - self.d * h + 2 * N_DEV, N_DEV)

        rings = [
            Ring(+1, rbuf_r, sbuf_r, ssem_r, rsem_r, credit_r, osem_r,
                 dst=right, sender=left, row_off=0),
            Ring(-1, rbuf_l, sbuf_l, ssem_l, rsem_l, credit_l, osem_l,
                 dst=left, sender=right, row_off=HH),
        ]

        for r in rings:
            r.seed()

        gc = 0.7978845608028654

        def gelu(y):
            return 0.5 * y * (1.0 + jnp.tanh(gc * (y + 0.044715 * y * y * y)))

        for r in rings:
            r.fill_sbuf(0, r.partial(i).astype(jnp.bfloat16))
        totals = [None, None]
        for s in range(N_DEV - 1):
            slot = s % 2
            rdmas = [r.send(s) for r in rings]
            parts = [r.partial(r.rs_recv_chunk(s)) for r in rings]
            for k, r in enumerate(rings):
                rdmas[k].wait()
                totals[k] = parts[k] + r.rbuf[slot, :, :]
                if s < N_DEV - 2:
                    r.fill_sbuf((s + 1) % 2, totals[k])
                r.give_credit()

        for k, r in enumerate(rings):
            y = totals[k].astype(jnp.float32)
            r.fill_sbuf(1, gelu(y).astype(jnp.bfloat16))
            r.store_out(1, r.own_chunk())

        for h in range(N_DEV - 1):
            t = N_DEV - 1 + h
            slot = t % 2
            nslot = (t + 1) % 2
            rdmas = [r.send(t) for r in rings]
            for k, r in enumerate(rings):
                rdmas[k].wait()
                r.fill_sbuf(nslot, r.rbuf[slot, :, :])
                r.give_credit()
                r.store_out(nslot, r.ag_recv_chunk(h))

        for r in rings:
            pl.semaphore_wait(r.credit, 2)
        for r in rings:
            for slot in (0, 1):
                if r.pending[slot] is not None:
                    r.pending[slot].wait()
                    r.pending[slot] = None

    ring_scratch = [
        pltpu.VMEM((2, HH, n), jnp.bfloat16),
        pltpu.VMEM((2, HH, n), jnp.bfloat16),
        pltpu.SemaphoreType.DMA((2,)),
        pltpu.SemaphoreType.DMA((2,)),
        pltpu.SemaphoreType.REGULAR,
        pltpu.SemaphoreType.DMA((2,)),
    ]
    return pl.pallas_call(
        body,
        out_shape=jax.ShapeDtypeStruct((m, n), jnp.bfloat16),
        in_specs=[pl.BlockSpec(memory_space=pltpu.VMEM),
                  pl.BlockSpec(memory_space=pltpu.VMEM)],
        out_specs=pl.BlockSpec(memory_space=pl.ANY),
        scratch_shapes=ring_scratch + ring_scratch,
        compiler_params=pltpu.CompilerParams(
            collective_id=0, vmem_limit_bytes=100 * 1024 * 1024),
    )(x, w_mat)


# device time: 747253 ns/iter; 1.8886x vs baseline; 1.0083x over previous
import jax
import jax.numpy as jnp
from jax import lax
from jax.experimental import pallas as pl
from jax.experimental.pallas import tpu as pltpu

N_DEV = 8
CH = 512
HH = CH // 2
SEG = 2
SH = HH // SEG


def kernel(x, w_mat):
    m, _ = x.shape
    _, n = w_mat.shape

    x = x.astype(jnp.bfloat16)
    w_mat = w_mat.astype(jnp.bfloat16)

    def body(x_ref, w_ref, out_ref,
             rbuf_r, sbuf_r, ssem_r, rsem_r, credit_r, osem_r,
             rbuf_l, sbuf_l, ssem_l, rsem_l, credit_l, osem_l):
        i = lax.axis_index("i")
        left = lax.rem(i - 1 + N_DEV, N_DEV)
        right = lax.rem(i + 1, N_DEV)

        barrier = pltpu.get_barrier_semaphore()
        for nbr in (left, right):
            pl.semaphore_signal(barrier, 1, device_id=(nbr,),
                                device_id_type=pl.DeviceIdType.MESH)
        pl.semaphore_wait(barrier, 2)

        class Ring:

            def __init__(self, d, rbuf, sbuf, ssem, rsem, credit, osem,
                         dst, sender, row_off):
                self.d = d
                self.rbuf, self.sbuf = rbuf, sbuf
                self.ssem, self.rsem = ssem, rsem
                self.credit, self.osem = credit, osem
                self.dst, self.sender = dst, sender
                self.row_off = row_off
                self.psend = [[None] * SEG, [None] * SEG]
                self.pstore = [[None] * SEG, [None] * SEG]

            def seed(self):
                pl.semaphore_signal(self.credit, 2 * SEG,
                                    device_id=(self.sender,),
                                    device_id_type=pl.DeviceIdType.MESH)

            def give_credit(self):
                pl.semaphore_signal(self.credit, 1, device_id=(self.sender,),
                                    device_id_type=pl.DeviceIdType.MESH)

            def partial(self, c):
                rows = pl.ds(c * CH + self.row_off, HH)
                return jnp.dot(x_ref[rows, :], w_ref[:, :],
                               preferred_element_type=jnp.float32
                               ).astype(jnp.bfloat16)

            def send_seg(self, step, seg):
                slot = step % 2
                pl.semaphore_wait(self.credit, 1)
                rdma = pltpu.make_async_remote_copy(
                    src_ref=self.sbuf.at[slot, seg],
                    dst_ref=self.rbuf.at[slot, seg],
                    send_sem=self.ssem.at[slot, seg],
                    recv_sem=self.rsem.at[slot, seg],
                    device_id=(self.dst,),
                    device_id_type=pl.DeviceIdType.MESH)
                rdma.start()
                self.psend[slot][seg] = rdma
                return rdma

            def wait_sbuf_free(self, slot, seg):
                if self.psend[slot][seg] is not None:
                    self.psend[slot][seg].wait_send()
                    self.psend[slot][seg] = None
                if self.pstore[slot][seg] is not None:
                    self.pstore[slot][seg].wait()
                    self.pstore[slot][seg] = None

            def store_out(self, slot, seg, c):
                cp = pltpu.make_async_copy(
                    self.sbuf.at[slot, seg],
                    out_ref.at[pl.ds(c * CH + self.row_off + seg * SH, SH), :],
                    self.osem.at[slot, seg])
                cp.start()
                self.pstore[slot][seg] = cp

            def drain(self):
                pl.semaphore_wait(self.credit, 2 * SEG)
                for slot in (0, 1):
                    for seg in range(SEG):
                        if self.psend[slot][seg] is not None:
                            self.psend[slot][seg].wait_send()
                            self.psend[slot][seg] = None
                        if self.pstore[slot][seg] is not None:
                            self.pstore[slot][seg].wait()
                            self.pstore[slot][seg] = None

            def rs_recv_chunk(self, s):
                return lax.rem(i - self.d * (1 + s) + 2 * N_DEV, N_DEV)

            def own_chunk(self):
                return lax.rem(i + self.d + N_DEV, N_DEV)

            def ag_recv_chunk(self, h):
                return lax.rem(i - self.d * h + 2 * N_DEV, N_DEV)

        rings = [
            Ring(+1, rbuf_r, sbuf_r, ssem_r, rsem_r, credit_r, osem_r,
                 dst=right, sender=left, row_off=0),
            Ring(-1, rbuf_l, sbuf_l, ssem_l, rsem_l, credit_l, osem_l,
                 dst=left, sender=right, row_off=HH),
        ]

        for r in rings:
            r.seed()

        gc = 0.7978845608028654

        def gelu(y):
            return 0.5 * y * (1.0 + jnp.tanh(gc * (y + 0.044715 * y * y * y)))

        def seg_rows(seg):
            return slice(seg * SH, (seg + 1) * SH)

        for r in rings:
            p = r.partial(i)
            for seg in range(SEG):
                r.sbuf[0, seg, :, :] = p[seg_rows(seg), :]

        totals = [[None] * SEG, [None] * SEG]
        for s in range(N_DEV - 1):
            slot = s % 2
            nslot = (s + 1) % 2
            rdmas = [[r.send_seg(s, seg) for seg in range(SEG)]
                     for r in rings]
            parts = [r.partial(r.rs_recv_chunk(s)) for r in rings]
            for seg in range(SEG):
                for k, r in enumerate(rings):
                    rdmas[k][seg].wait_recv()
                    acc = (parts[k][seg_rows(seg), :]
                           + r.rbuf[slot, seg, :, :])
                    if s < N_DEV - 2:
                        r.wait_sbuf_free(nslot, seg)
                        r.sbuf[nslot, seg, :, :] = acc
                    else:
                        totals[k][seg] = acc
                    r.give_credit()

        for k, r in enumerate(rings):
            c_own = r.own_chunk()
            for seg in range(SEG):
                y = totals[k][seg].astype(jnp.float32)
                r.wait_sbuf_free(1, seg)
                r.sbuf[1, seg, :, :] = gelu(y).astype(jnp.bfloat16)
                r.store_out(1, seg, c_own)

        for h in range(N_DEV - 1):
            t = N_DEV - 1 + h
            slot = t % 2
            nslot = (t + 1) % 2
            rdmas = [[r.send_seg(t, seg) for seg in range(SEG)]
                     for r in rings]
            for seg in range(SEG):
                for k, r in enumerate(rings):
                    rdmas[k][seg].wait_recv()
                    c = r.ag_recv_chunk(h)
                    r.wait_sbuf_free(nslot, seg)
                    r.sbuf[nslot, seg, :, :] = r.rbuf[slot, seg, :, :]
                    r.give_credit()
                    r.store_out(nslot, seg, c)

        for r in rings:
            r.drain()

    ring_scratch = [
        pltpu.VMEM((2, SEG, SH, n), jnp.bfloat16),
        pltpu.VMEM((2, SEG, SH, n), jnp.bfloat16),
        pltpu.SemaphoreType.DMA((2, SEG)),
        pltpu.SemaphoreType.DMA((2, SEG)),
        pltpu.SemaphoreType.REGULAR,
        pltpu.SemaphoreType.DMA((2, SEG)),
    ]
    return pl.pallas_call(
        body,
        out_shape=jax.ShapeDtypeStruct((m, n), jnp.bfloat16),
        in_specs=[pl.BlockSpec(memory_space=pltpu.VMEM),
                  pl.BlockSpec(memory_space=pltpu.VMEM)],
        out_specs=pl.BlockSpec(memory_space=pl.ANY),
        scratch_shapes=ring_scratch + ring_scratch,
        compiler_params=pltpu.CompilerParams(
            collective_id=0, vmem_limit_bytes=100 * 1024 * 1024),
    )(x, w_mat)


# device time: 704761 ns/iter; 2.0024x vs baseline; 1.0603x over previous
import jax
import jax.numpy as jnp
from jax import lax
from jax.experimental import pallas as pl
from jax.experimental.pallas import tpu as pltpu

N_DEV = 8
CH = 512
HH = CH // 2
SEG = 2
SH = HH // SEG


def kernel(x, w_mat):
    m, _ = x.shape
    _, n = w_mat.shape

    x = x.astype(jnp.bfloat16)
    w_mat = w_mat.astype(jnp.bfloat16)

    def body(x_ref, w_ref, out_ref,
             rbuf_r, sbuf_r, ssem_r, rsem_r, credit_r, osem_r,
             rbuf_l, sbuf_l, ssem_l, rsem_l, credit_l, osem_l):
        i = lax.axis_index("i")
        left = lax.rem(i - 1 + N_DEV, N_DEV)
        right = lax.rem(i + 1, N_DEV)

        barrier = pltpu.get_barrier_semaphore()
        for nbr in (left, right):
            pl.semaphore_signal(barrier, 1, device_id=(nbr,),
                                device_id_type=pl.DeviceIdType.MESH)
        pl.semaphore_wait(barrier, 2)

        class Ring:

            def __init__(self, d, rbuf, sbuf, ssem, rsem, credit, osem,
                         dst, sender, row_off):
                self.d = d
                self.rbuf, self.sbuf = rbuf, sbuf
                self.ssem, self.rsem = ssem, rsem
                self.credit, self.osem = credit, osem
                self.dst, self.sender = dst, sender
                self.row_off = row_off
                self.inflight = [[None] * SEG, [None] * SEG]
                self.swaited = [[True] * SEG, [True] * SEG]
                self.pstore = [[None] * SEG, [None] * SEG]

            def seed(self):
                pl.semaphore_signal(self.credit, 2 * SEG,
                                    device_id=(self.sender,),
                                    device_id_type=pl.DeviceIdType.MESH)

            def give_credit(self):
                pl.semaphore_signal(self.credit, 1, device_id=(self.sender,),
                                    device_id_type=pl.DeviceIdType.MESH)

            def partial(self, c):
                rows = pl.ds(c * CH + self.row_off, HH)
                return jnp.dot(x_ref[rows, :], w_ref[:, :],
                               preferred_element_type=jnp.float32
                               ).astype(jnp.bfloat16)

            def send_seg(self, step, seg):
                slot = step % 2
                assert self.swaited[slot][seg]
                pl.semaphore_wait(self.credit, 1)
                rdma = pltpu.make_async_remote_copy(
                    src_ref=self.sbuf.at[slot, seg],
                    dst_ref=self.rbuf.at[slot, seg],
                    send_sem=self.ssem.at[slot, seg],
                    recv_sem=self.rsem.at[slot, seg],
                    device_id=(self.dst,),
                    device_id_type=pl.DeviceIdType.MESH)
                rdma.start()
                self.inflight[slot][seg] = rdma
                self.swaited[slot][seg] = False

            def wait_recv(self, step, seg):
                self.inflight[step % 2][seg].wait_recv()

            def wait_sbuf_free(self, slot, seg):
                if (self.inflight[slot][seg] is not None
                        and not self.swaited[slot][seg]):
                    self.inflight[slot][seg].wait_send()
                    self.swaited[slot][seg] = True
                if self.pstore[slot][seg] is not None:
                    self.pstore[slot][seg].wait()
                    self.pstore[slot][seg] = None

            def store_out(self, slot, seg, c):
                cp = pltpu.make_async_copy(
                    self.sbuf.at[slot, seg],
                    out_ref.at[pl.ds(c * CH + self.row_off + seg * SH, SH), :],
                    self.osem.at[slot, seg])
                cp.start()
                self.pstore[slot][seg] = cp

            def drain(self):
                pl.semaphore_wait(self.credit, 2 * SEG)
                for slot in (0, 1):
                    for seg in range(SEG):
                        self.wait_sbuf_free(slot, seg)

            def rs_recv_chunk(self, s):
                return lax.rem(i - self.d * (1 + s) + 2 * N_DEV, N_DEV)

            def own_chunk(self):
                return lax.rem(i + self.d + N_DEV, N_DEV)

            def ag_recv_chunk(self, h):
                return lax.rem(i - self.d * h + 2 * N_DEV, N_DEV)

        rings = [
            Ring(+1, rbuf_r, sbuf_r, ssem_r, rsem_r, credit_r, osem_r,
                 dst=right, sender=left, row_off=0),
            Ring(-1, rbuf_l, sbuf_l, ssem_l, rsem_l, credit_l, osem_l,
                 dst=left, sender=right, row_off=HH),
        ]

        for r in rings:
            r.seed()

        gc = 0.7978845608028654

        def gelu(y):
            return 0.5 * y * (1.0 + jnp.tanh(gc * (y + 0.044715 * y * y * y)))

        def seg_rows(seg):
            return slice(seg * SH, (seg + 1) * SH)

        for r in rings:
            p = r.partial(i)
            for seg in range(SEG):
                r.sbuf[0, seg, :, :] = p[seg_rows(seg), :]
        for seg in range(SEG):
            for r in rings:
                r.send_seg(0, seg)
        parts = [r.partial(r.rs_recv_chunk(0)) for r in rings]

        for s in range(N_DEV - 1):
            slot, ns = s % 2, (s + 1) % 2
            last = s == N_DEV - 2
            for seg in range(SEG):
                for k, r in enumerate(rings):
                    r.wait_recv(s, seg)
                    acc = parts[k][seg_rows(seg), :] + r.rbuf[slot, seg, :, :]
                    r.wait_sbuf_free(ns, seg)
                    if last:
                        y = gelu(acc.astype(jnp.float32))
                        r.sbuf[ns, seg, :, :] = y.astype(jnp.bfloat16)
                        r.give_credit()
                        r.store_out(ns, seg, r.own_chunk())
                    else:
                        r.sbuf[ns, seg, :, :] = acc
                        r.give_credit()
                for r in rings:
                    r.send_seg(s + 1, seg)
            if not last:
                parts = [r.partial(r.rs_recv_chunk(s + 1)) for r in rings]

        for h in range(N_DEV - 1):
            t = N_DEV - 1 + h
            slot, ns = t % 2, (t + 1) % 2
            for seg in range(SEG):
                for k, r in enumerate(rings):
                    r.wait_recv(t, seg)
                    r.wait_sbuf_free(ns, seg)
                    r.sbuf[ns, seg, :, :] = r.rbuf[slot, seg, :, :]
                    r.give_credit()
                    r.store_out(ns, seg, r.ag_recv_chunk(h))
                if h < N_DEV - 2:
                    for r in rings:
                        r.send_seg(t + 1, seg)

        for r in rings:
            r.drain()

    ring_scratch = [
        pltpu.VMEM((2, SEG, SH, n), jnp.bfloat16),
        pltpu.VMEM((2, SEG, SH, n), jnp.bfloat16),
        pltpu.SemaphoreType.DMA((2, SEG)),
        pltpu.SemaphoreType.DMA((2, SEG)),
        pltpu.SemaphoreType.REGULAR,
        pltpu.SemaphoreType.DMA((2, SEG)),
    ]
    return pl.pallas_call(
        body,
        out_shape=jax.ShapeDtypeStruct((m, n), jnp.bfloat16),
        in_specs=[pl.BlockSpec(memory_space=pltpu.VMEM),
                  pl.BlockSpec(memory_space=pltpu.VMEM)],
        out_specs=pl.BlockSpec(memory_space=pl.ANY),
        scratch_shapes=ring_scratch + ring_scratch,
        compiler_params=pltpu.CompilerParams(
            collective_id=0, vmem_limit_bytes=100 * 1024 * 1024),
    )(x, w_mat)
